# baseline (speedup 1.0000x reference)
import functools

import jax
import jax.numpy as jnp
from jax import lax
from jax.experimental import pallas as pl
from jax.experimental.pallas import tpu as pltpu
from jax.experimental.pallas import tpu_sc as plsc

_SC_PARAMS = pltpu.CompilerParams(
    use_tc_tiling_on_sc=True, needs_layout_passes=False)


def _relayout_lines(tabT, tail_lines, V, D, NC, NW, mesh):
    NBLK = V // 128
    TAIL = V - NBLK * 128
    PER = -(-NBLK // NW)

    @functools.partial(
        pl.kernel,
        mesh=mesh,
        out_type=jax.ShapeDtypeStruct((-(-V // 4 // 8) * 8, 4 * D),
                                      jnp.float32),
        scratch_types=[
            pltpu.VMEM((D, 128), jnp.float32),
            pltpu.VMEM((D, 128), jnp.float32),
            pltpu.VMEM((32, 4 * D), jnp.float32),
            pltpu.SemaphoreType.DMA,
            pltpu.SemaphoreType.DMA,
        ],
        compiler_params=_SC_PARAMS,
    )
    def run(tab_hbm, tail_hbm, lin_hbm, inA, inB, o_v, semA, semB):
        w = lax.axis_index("s") * NC + lax.axis_index("c")
        n_w = jnp.minimum(PER, NBLK - w * PER)

        def fire(u, buf, sem):
            c = w * PER + u
            pltpu.async_copy(tab_hbm.at[:, pl.ds(c * 128, 128)], buf, sem)

        def drain(u, buf, sem):
            c = w * PER + u
            pltpu.make_async_copy(
                tab_hbm.at[:, pl.ds(c * 128, 128)], buf, sem).wait()

        def transpose_block(buf, nrows):
            for w2 in range(nrows):
                for g in range(8):
                    rows = lax.iota(jnp.int32, 16) + 16 * (g % 2)
                    cols = jnp.full((16,), 4 * w2 + g // 2, jnp.int32)
                    o_v[w2, pl.ds(16 * g, 16)] = plsc.load_gather(
                        buf, [rows, cols])

        def emit(u, buf):
            transpose_block(buf, 32)
            c = w * PER + u
            pltpu.sync_copy(o_v, lin_hbm.at[pl.ds(c * 32, 32)])

        @pl.when(n_w > 0)
        def _():
            fire(0, inA, semA)

        def body(m, carry):
            u = 2 * m

            @pl.when(u + 1 < n_w)
            def _():
                fire(u + 1, inB, semB)

            drain(u, inA, semA)
            emit(u, inA)

            @pl.when(u + 2 < n_w)
            def _():
                fire(u + 2, inA, semA)

            @pl.when(u + 1 < n_w)
            def _():
                drain(u + 1, inB, semB)
                emit(u + 1, inB)

            return carry

        lax.fori_loop(0, (n_w + 1) // 2, body, 0)

        if TAIL:
            @pl.when(w == NW - 1)
            def _():
                pltpu.sync_copy(tail_hbm, o_v.at[pl.ds(0, TAIL // 4)])
                pltpu.sync_copy(o_v.at[pl.ds(0, TAIL // 4)],
                                lin_hbm.at[pl.ds(NBLK * 32, TAIL // 4)])

    return run(tabT, tail_lines)


def kernel(indices, table):
    B, T = indices.shape
    V, D = table.shape
    idxT = indices.T.astype(jnp.int32)
    tabT = table.T

    info = plsc.get_sparse_core_info()
    NC, NS = info.num_cores, info.num_subcores
    NW = NC * NS
    assert B == NW * 128 and V % 8 == 0 and D == 32
    mesh = plsc.VectorSubcoreMesh(core_axis_name="c", subcore_axis_name="s")

    tail = V % 128
    tail_lines = (table[V - tail:].reshape(tail // 4, 4 * D) if tail
                  else jnp.zeros((8, 4 * D), jnp.float32))
    lines = _relayout_lines(tabT, tail_lines, V, D, NC, NW, mesh)

    @functools.partial(
        pl.kernel,
        mesh=mesh,
        out_type=jax.ShapeDtypeStruct((T, D, B), jnp.float32),
        scratch_types=[
            pltpu.VMEM((T, 128), jnp.int32),
            pltpu.VMEM((128,), jnp.int32),
            pltpu.VMEM((128,), jnp.int32),
            pltpu.VMEM((128,), jnp.int32),
            pltpu.VMEM((128,), jnp.int32),
            pltpu.VMEM((128, 4 * D), jnp.float32),
            pltpu.VMEM((128, 4 * D), jnp.float32),
            pltpu.VMEM((D, 128), jnp.float32),
            pltpu.SemaphoreType.DMA,
            pltpu.SemaphoreType.DMA,
        ],
        compiler_params=_SC_PARAMS,
    )
    def run(idx_hbm, lin_hbm, out_hbm, idx_v, lnA, lnB, qcA, qcB, gA, gB, o_v,
            semA, semB):
        w = lax.axis_index("s") * NC + lax.axis_index("c")
        pltpu.sync_copy(idx_hbm.at[:, pl.ds(w * 128, 128)], idx_v)

        def prep(t, ln, qc):
            for g in range(8):
                v = idx_v[t, pl.ds(16 * g, 16)]
                qc[pl.ds(16 * g, 16)] = (v & 3) * D
                ln[pl.ds(16 * g, 16)] = v >> 2

        def extract(gbuf, qc, t):
            for g in range(8):
                rows = lax.iota(jnp.int32, 16) + 16 * g
                qcg = qc[pl.ds(16 * g, 16)]
                for e in range(D):
                    o_v[e, pl.ds(16 * g, 16)] = plsc.load_gather(
                        gbuf, [rows, qcg + e])
            pltpu.sync_copy(o_v, out_hbm.at[t, :, pl.ds(w * 128, 128)])

        prep(0, lnA, qcA)
        pltpu.async_copy(lin_hbm.at[lnA], gA, semA)

        def body(m, carry):
            t = 2 * m
            prep(t + 1, lnB, qcB)
            pltpu.async_copy(lin_hbm.at[lnB], gB, semB)
            pltpu.make_async_copy(lin_hbm.at[lnA], gA, semA).wait()
            extract(gA, qcA, t)

            @pl.when(t + 2 < T)
            def _():
                prep(t + 2, lnA, qcA)
                pltpu.async_copy(lin_hbm.at[lnA], gA, semA)

            pltpu.make_async_copy(lin_hbm.at[lnB], gB, semB).wait()
            extract(gB, qcB, t + 1)
            return carry

        lax.fori_loop(0, T // 2, body, 0)

    outT = run(idxT, lines)
    return outT.transpose(2, 0, 1)


# speedup vs baseline: 1.9554x; 1.9554x over previous
import functools

import jax
import jax.numpy as jnp
from jax import lax
from jax.experimental import pallas as pl
from jax.experimental.pallas import tpu as pltpu
from jax.experimental.pallas import tpu_sc as plsc

_SC_PARAMS = pltpu.CompilerParams(
    use_tc_tiling_on_sc=True, needs_layout_passes=False)


def _relayout_lines(tabT, tail_lines, V, D, NC, NW, mesh):
    NBLK = V // 128
    TAIL = V - NBLK * 128
    PER = -(-NBLK // NW)

    @functools.partial(
        pl.kernel,
        mesh=mesh,
        out_type=jax.ShapeDtypeStruct((-(-V // 4 // 8) * 8, 4 * D),
                                      jnp.float32),
        scratch_types=[
            pltpu.VMEM((D, 128), jnp.float32),
            pltpu.VMEM((D, 128), jnp.float32),
            pltpu.VMEM((32, 4 * D), jnp.float32),
            pltpu.SemaphoreType.DMA,
            pltpu.SemaphoreType.DMA,
        ],
        compiler_params=_SC_PARAMS,
    )
    def run(tab_hbm, tail_hbm, lin_hbm, inA, inB, o_v, semA, semB):
        w = lax.axis_index("s") * NC + lax.axis_index("c")
        n_w = jnp.minimum(PER, NBLK - w * PER)

        def fire(u, buf, sem):
            c = w * PER + u
            pltpu.async_copy(tab_hbm.at[:, pl.ds(c * 128, 128)], buf, sem)

        def drain(u, buf, sem):
            c = w * PER + u
            pltpu.make_async_copy(
                tab_hbm.at[:, pl.ds(c * 128, 128)], buf, sem).wait()

        def transpose_block(buf, nrows):
            rows = [lax.iota(jnp.int32, 16) + 16 * h for h in range(2)]

            @plsc.parallel_loop(0, nrows, unroll=8)
            def _(w2):
                cols = [jnp.full((16,), 4 * w2 + q, jnp.int32)
                        for q in range(4)]
                for g in range(8):
                    o_v[w2, pl.ds(16 * g, 16)] = plsc.load_gather(
                        buf, [rows[g % 2], cols[g // 2]])

        def emit(u, buf):
            transpose_block(buf, 32)
            c = w * PER + u
            pltpu.sync_copy(o_v, lin_hbm.at[pl.ds(c * 32, 32)])

        @pl.when(n_w > 0)
        def _():
            fire(0, inA, semA)

        def body(m, carry):
            u = 2 * m

            @pl.when(u + 1 < n_w)
            def _():
                fire(u + 1, inB, semB)

            drain(u, inA, semA)
            emit(u, inA)

            @pl.when(u + 2 < n_w)
            def _():
                fire(u + 2, inA, semA)

            @pl.when(u + 1 < n_w)
            def _():
                drain(u + 1, inB, semB)
                emit(u + 1, inB)

            return carry

        lax.fori_loop(0, (n_w + 1) // 2, body, 0)

        if TAIL:
            @pl.when(w == NW - 1)
            def _():
                pltpu.sync_copy(tail_hbm, o_v.at[pl.ds(0, TAIL // 4)])
                pltpu.sync_copy(o_v.at[pl.ds(0, TAIL // 4)],
                                lin_hbm.at[pl.ds(NBLK * 32, TAIL // 4)])

    return run(tabT, tail_lines)


def kernel(indices, table):
    B, T = indices.shape
    V, D = table.shape
    idxT = indices.T.astype(jnp.int32)
    tabT = table.T

    info = plsc.get_sparse_core_info()
    NC, NS = info.num_cores, info.num_subcores
    NW = NC * NS
    assert B == NW * 128 and V % 8 == 0 and D == 32
    mesh = plsc.VectorSubcoreMesh(core_axis_name="c", subcore_axis_name="s")

    tail = V % 128
    tail_lines = (table[V - tail:].reshape(tail // 4, 4 * D) if tail
                  else jnp.zeros((8, 4 * D), jnp.float32))
    lines = _relayout_lines(tabT, tail_lines, V, D, NC, NW, mesh)

    @functools.partial(
        pl.kernel,
        mesh=mesh,
        out_type=jax.ShapeDtypeStruct((T, D, B), jnp.float32),
        scratch_types=[
            pltpu.VMEM((T, 128), jnp.int32),
            pltpu.VMEM((128,), jnp.int32),
            pltpu.VMEM((128,), jnp.int32),
            pltpu.VMEM((128,), jnp.int32),
            pltpu.VMEM((128,), jnp.int32),
            pltpu.VMEM((128, 4 * D), jnp.float32),
            pltpu.VMEM((128, 4 * D), jnp.float32),
            pltpu.VMEM((D, 128), jnp.float32),
            pltpu.SemaphoreType.DMA,
            pltpu.SemaphoreType.DMA,
        ],
        compiler_params=_SC_PARAMS,
    )
    def run(idx_hbm, lin_hbm, out_hbm, idx_v, lnA, lnB, qcA, qcB, gA, gB, o_v,
            semA, semB):
        w = lax.axis_index("s") * NC + lax.axis_index("c")
        pltpu.sync_copy(idx_hbm.at[:, pl.ds(w * 128, 128)], idx_v)

        def prep(t, ln, qc):
            for g in range(8):
                v = idx_v[t, pl.ds(16 * g, 16)]
                qc[pl.ds(16 * g, 16)] = (v & 3) * D
                ln[pl.ds(16 * g, 16)] = v >> 2

        def extract(gbuf, qc, t):
            rows = [lax.iota(jnp.int32, 16) + 16 * g for g in range(8)]
            qcs = [qc[pl.ds(16 * g, 16)] for g in range(8)]

            @plsc.parallel_loop(0, D, unroll=8)
            def _(e):
                for g in range(8):
                    o_v[e, pl.ds(16 * g, 16)] = plsc.load_gather(
                        gbuf, [rows[g], qcs[g] + e])

            pltpu.sync_copy(o_v, out_hbm.at[t, :, pl.ds(w * 128, 128)])

        prep(0, lnA, qcA)
        pltpu.async_copy(lin_hbm.at[lnA], gA, semA)

        def body(m, carry):
            t = 2 * m
            prep(t + 1, lnB, qcB)
            pltpu.async_copy(lin_hbm.at[lnB], gB, semB)
            pltpu.make_async_copy(lin_hbm.at[lnA], gA, semA).wait()
            extract(gA, qcA, t)

            @pl.when(t + 2 < T)
            def _():
                prep(t + 2, lnA, qcA)
                pltpu.async_copy(lin_hbm.at[lnA], gA, semA)

            pltpu.make_async_copy(lin_hbm.at[lnB], gB, semB).wait()
            extract(gB, qcB, t + 1)
            return carry

        lax.fori_loop(0, T // 2, body, 0)

    outT = run(idxT, lines)
    return outT.transpose(2, 0, 1)


# speedup vs baseline: 2.1584x; 1.1038x over previous
import functools

import jax
import jax.numpy as jnp
from jax import lax
from jax.experimental import pallas as pl
from jax.experimental.pallas import tpu as pltpu
from jax.experimental.pallas import tpu_sc as plsc

_SC_PARAMS = pltpu.CompilerParams(
    use_tc_tiling_on_sc=True, needs_layout_passes=False)

_CW = 256


def _relayout_lines(tabT, tail_lines, V, D, NC, NW, mesh):
    NBLK = V // _CW
    TAIL = V - NBLK * _CW
    LW = _CW // 4
    PER = -(-NBLK // NW)
    NLINES = -(-(V // 4) // 8) * 8

    @functools.partial(
        pl.kernel,
        mesh=mesh,
        out_type=jax.ShapeDtypeStruct((NLINES, 4 * D), jnp.float32),
        scratch_types=[
            pltpu.VMEM((D, _CW), jnp.float32),
            pltpu.VMEM((D, _CW), jnp.float32),
            pltpu.VMEM((LW, 4 * D), jnp.float32),
            pltpu.VMEM((LW, 4 * D), jnp.float32),
            pltpu.SemaphoreType.DMA,
            pltpu.SemaphoreType.DMA,
            pltpu.SemaphoreType.DMA,
            pltpu.SemaphoreType.DMA,
        ],
        compiler_params=_SC_PARAMS,
    )
    def run(tab_hbm, tail_hbm, lin_hbm, inA, inB, ovA, ovB, siA, siB, soA,
            soB):
        w = lax.axis_index("s") * NC + lax.axis_index("c")
        n_w = jnp.minimum(PER, NBLK - w * PER)

        def fire_in(u, buf, sem):
            c = w * PER + u
            pltpu.async_copy(tab_hbm.at[:, pl.ds(c * _CW, _CW)], buf, sem)

        def drain_in(u, buf, sem):
            c = w * PER + u
            pltpu.make_async_copy(
                tab_hbm.at[:, pl.ds(c * _CW, _CW)], buf, sem).wait()

        def fire_out(u, ov, sem):
            c = w * PER + u
            pltpu.async_copy(ov, lin_hbm.at[pl.ds(c * LW, LW)], sem)

        def drain_out(u, ov, sem):
            c = w * PER + u
            pltpu.make_async_copy(
                ov, lin_hbm.at[pl.ds(c * LW, LW)], sem).wait()

        def transpose_block(buf, ov):
            rows = [lax.iota(jnp.int32, 16) + 16 * h for h in range(2)]

            @plsc.parallel_loop(0, LW, unroll=16)
            def _(w2):
                cols = [jnp.full((16,), 4 * w2 + q, jnp.int32)
                        for q in range(4)]
                for g in range(8):
                    ov[w2, pl.ds(16 * g, 16)] = plsc.load_gather(
                        buf, [rows[g % 2], cols[g // 2]])

        @pl.when(n_w > 0)
        def _():
            fire_in(0, inA, siA)

        def body(m, carry):
            u = 2 * m

            @pl.when(u + 1 < n_w)
            def _():
                fire_in(u + 1, inB, siB)

            drain_in(u, inA, siA)

            @pl.when(m > 0)
            def _():
                drain_out(u - 2, ovA, soA)

            transpose_block(inA, ovA)
            fire_out(u, ovA, soA)

            @pl.when(u + 2 < n_w)
            def _():
                fire_in(u + 2, inA, siA)

            @pl.when(u + 1 < n_w)
            def _():
                drain_in(u + 1, inB, siB)

                @pl.when(m > 0)
                def _():
                    drain_out(u - 1, ovB, soB)

                transpose_block(inB, ovB)
                fire_out(u + 1, ovB, soB)

            return carry

        lax.fori_loop(0, (n_w + 1) // 2, body, 0)

        @pl.when(n_w > 0)
        def _():
            drain_out(2 * ((n_w - 1) // 2), ovA, soA)

        @pl.when(n_w > 1)
        def _():
            drain_out(2 * (n_w // 2) - 1, ovB, soB)

        if TAIL:
            @pl.when(w == NW - 1)
            def _():
                pltpu.sync_copy(tail_hbm, ovA.at[pl.ds(0, TAIL // 4)])
                pltpu.sync_copy(ovA.at[pl.ds(0, TAIL // 4)],
                                lin_hbm.at[pl.ds(NBLK * LW, TAIL // 4)])

    return run(tabT, tail_lines)


def kernel(indices, table):
    B, T = indices.shape
    V, D = table.shape
    idxT = indices.T.astype(jnp.int32)
    tabT = table.T

    info = plsc.get_sparse_core_info()
    NC, NS = info.num_cores, info.num_subcores
    NW = NC * NS
    assert B == NW * 128 and D == 32 and T % 2 == 0
    tail = V % _CW
    assert tail % 32 == 0
    mesh = plsc.VectorSubcoreMesh(core_axis_name="c", subcore_axis_name="s")

    tail_lines = (table[V - tail:].reshape(tail // 4, 4 * D) if tail
                  else jnp.zeros((8, 4 * D), jnp.float32))
    lines = _relayout_lines(tabT, tail_lines, V, D, NC, NW, mesh)

    @functools.partial(
        pl.kernel,
        mesh=mesh,
        out_type=jax.ShapeDtypeStruct((T, D, B), jnp.float32),
        scratch_types=[
            pltpu.VMEM((T, 128), jnp.int32),
            pltpu.VMEM((128,), jnp.int32),
            pltpu.VMEM((128,), jnp.int32),
            pltpu.VMEM((128,), jnp.int32),
            pltpu.VMEM((128,), jnp.int32),
            pltpu.VMEM((128, 4 * D), jnp.float32),
            pltpu.VMEM((128, 4 * D), jnp.float32),
            pltpu.VMEM((D, 128), jnp.float32),
            pltpu.VMEM((D, 128), jnp.float32),
            pltpu.SemaphoreType.DMA,
            pltpu.SemaphoreType.DMA,
            pltpu.SemaphoreType.DMA,
            pltpu.SemaphoreType.DMA,
        ],
        compiler_params=_SC_PARAMS,
    )
    def run(idx_hbm, lin_hbm, out_hbm, idx_v, lnA, lnB, qcA, qcB, gA, gB,
            ovA, ovB, sgA, sgB, soA, soB):
        w = lax.axis_index("s") * NC + lax.axis_index("c")
        pltpu.sync_copy(idx_hbm.at[:, pl.ds(w * 128, 128)], idx_v)

        def prep(t, ln, qc):
            for g in range(8):
                v = idx_v[t, pl.ds(16 * g, 16)]
                qc[pl.ds(16 * g, 16)] = (v & 3) * D
                ln[pl.ds(16 * g, 16)] = v >> 2

        def fire_out(t, ov, sem):
            pltpu.async_copy(ov, out_hbm.at[t, :, pl.ds(w * 128, 128)], sem)

        def drain_out(t, ov, sem):
            pltpu.make_async_copy(
                ov, out_hbm.at[t, :, pl.ds(w * 128, 128)], sem).wait()

        def extract(gbuf, qc, ov):
            rows = [lax.iota(jnp.int32, 16) + 16 * g for g in range(8)]
            qcs = [qc[pl.ds(16 * g, 16)] for g in range(8)]

            @plsc.parallel_loop(0, D, unroll=8)
            def _(e):
                for g in range(8):
                    ov[e, pl.ds(16 * g, 16)] = plsc.load_gather(
                        gbuf, [rows[g], qcs[g] + e])

        prep(0, lnA, qcA)
        pltpu.async_copy(lin_hbm.at[lnA], gA, sgA)

        def body(m, carry):
            t = 2 * m
            prep(t + 1, lnB, qcB)
            pltpu.async_copy(lin_hbm.at[lnB], gB, sgB)
            pltpu.make_async_copy(lin_hbm.at[lnA], gA, sgA).wait()

            @pl.when(m > 0)
            def _():
                drain_out(t - 2, ovA, soA)

            extract(gA, qcA, ovA)
            fire_out(t, ovA, soA)

            @pl.when(t + 2 < T)
            def _():
                prep(t + 2, lnA, qcA)
                pltpu.async_copy(lin_hbm.at[lnA], gA, sgA)

            pltpu.make_async_copy(lin_hbm.at[lnB], gB, sgB).wait()

            @pl.when(m > 0)
            def _():
                drain_out(t - 1, ovB, soB)

            extract(gB, qcB, ovB)
            fire_out(t + 1, ovB, soB)
            return carry

        lax.fori_loop(0, T // 2, body, 0)
        drain_out(T - 2, ovA, soA)
        drain_out(T - 1, ovB, soB)

    outT = run(idxT, lines)
    return outT.transpose(2, 0, 1)
